# baseline (speedup 1.0000x reference)
import functools

import jax
import jax.numpy as jnp
from jax import lax
from jax.experimental import pallas as pl
from jax.experimental.pallas import tpu as pltpu
from jax.experimental.pallas import tpu_sc as plsc

_SIZES = (1024, 512, 256, 128)
_C = 16
_B = 4
_HW = 512
_P = _B * _HW * _HW
_NW = 32
_PPW = _P // _NW
_CHUNK = 128
_NCHUNK = _PPW // _CHUNK
_NG = 16


def _sc_sample(uv, t0, t1, t2, t3):
    mesh = plsc.VectorSubcoreMesh(core_axis_name="c", subcore_axis_name="s")

    @functools.partial(
        pl.kernel,
        mesh=mesh,
        out_type=jax.ShapeDtypeStruct((_B, _HW, _HW, _C), jnp.float32),
        compiler_params=pltpu.CompilerParams(use_tc_tiling_on_sc=False),
        scratch_types=[
            pltpu.VMEM((2, _CHUNK), jnp.float32),
            pltpu.VMEM((_NG, _CHUNK), jnp.int32),
            pltpu.VMEM((_NG, _CHUNK), jnp.int32),
            pltpu.VMEM((_NG, _CHUNK), jnp.float32),
            pltpu.VMEM((_NG, _CHUNK), jnp.float32),
            pltpu.VMEM((_NG * _CHUNK, _C), jnp.float32),
            pltpu.VMEM((_NG * _CHUNK, _C), jnp.float32),
            pltpu.VMEM((_CHUNK, _C), jnp.float32),
            pltpu.VMEM((_CHUNK, _C), jnp.float32),
            pltpu.SemaphoreType.DMA,
            pltpu.SemaphoreType.DMA,
            pltpu.SemaphoreType.DMA,
        ],
    )
    def body(uv_hbm, t0_hbm, t1_hbm, t2_hbm, t3_hbm, out_hbm,
             uv_v, idxA, idxB, wA, wB, rowsA, rowsB, oA, oB, semA, semB, semO):
        tabs = (t0_hbm, t1_hbm, t2_hbm, t3_hbm)
        wid = lax.axis_index("s") * 2 + lax.axis_index("c")
        wbase = wid * _PPW

        def fire(g, idx_v, w_v, rows_v, sem):
            base = wbase + g * _CHUNK
            pltpu.sync_copy(uv_hbm.at[:, pl.ds(base, _CHUNK)], uv_v)

            def grp_body(gi, c2):
                sl = pl.ds(gi * 16, 16)
                uu = uv_v[0, sl]
                vv = uv_v[1, sl]
                for li, s in enumerate(_SIZES):
                    ix = ((2.0 * uu - 1.0 + 1.0) * s - 1.0) * 0.5
                    iy = ((2.0 * vv - 1.0 + 1.0) * s - 1.0) * 0.5
                    x0i = (ix + 1.0).astype(jnp.int32)
                    y0i = (iy + 1.0).astype(jnp.int32)
                    fx = ix - (x0i.astype(jnp.float32) - 1.0)
                    fy = iy - (y0i.astype(jnp.float32) - 1.0)
                    xc0 = jnp.maximum(x0i - 1, 0)
                    xc1 = jnp.minimum(jnp.maximum(x0i, 0), s - 1)
                    yc0 = jnp.maximum(y0i - 1, 0)
                    yc1 = jnp.minimum(jnp.maximum(y0i, 0), s - 1)
                    w0x = jnp.where(x0i >= 1, 1.0 - fx, 0.0)
                    w1x = jnp.where(x0i <= s - 1, fx, 0.0)
                    w0y = jnp.where(y0i >= 1, 1.0 - fy, 0.0)
                    w1y = jnp.where(y0i <= s - 1, fy, 0.0)
                    r0 = yc0 * s
                    r1 = yc1 * s
                    idx_v[li * 4 + 0, sl] = r0 + xc0
                    idx_v[li * 4 + 1, sl] = r0 + xc1
                    idx_v[li * 4 + 2, sl] = r1 + xc0
                    idx_v[li * 4 + 3, sl] = r1 + xc1
                    w_v[li * 4 + 0, sl] = w0x * w0y
                    w_v[li * 4 + 1, sl] = w1x * w0y
                    w_v[li * 4 + 2, sl] = w0x * w1y
                    w_v[li * 4 + 3, sl] = w1x * w1y
                return c2

            lax.fori_loop(0, _CHUNK // 16, grp_body, 0)
            for li in range(4):
                for c in range(4):
                    k = li * 4 + c
                    pltpu.async_copy(
                        tabs[li].at[idx_v.at[k]],
                        rows_v.at[pl.ds(k * _CHUNK, _CHUNK)], sem)

        def process(g, w_v, rows_v, sem, o_v):
            pltpu.make_async_copy(
                t0_hbm.at[pl.ds(0, _NG * _CHUNK)], rows_v, sem).wait()

            @pl.when(g >= 2)
            def _():
                pltpu.make_async_copy(
                    o_v, out_hbm.at[0, 0, pl.ds(0, _CHUNK)], semO).wait()

            def wgrp_body(gi, c3):
                sl = pl.ds(gi * 16, 16)
                wk = [w_v[k, sl] for k in range(_NG)]
                for j in range(16):
                    p = gi * 16 + j
                    acc = wk[0][j] * rows_v[p]
                    for k in range(1, _NG):
                        acc = acc + wk[k][j] * rows_v[k * _CHUNK + p]
                    o_v[p] = acc
                return c3

            lax.fori_loop(0, _CHUNK // 16, wgrp_body, 0)
            base = wbase + g * _CHUNK
            bb = base // (_HW * _HW)
            yy = (base // _HW) % _HW
            xx = base % _HW
            pltpu.async_copy(o_v, out_hbm.at[bb, yy, pl.ds(xx, _CHUNK)], semO)

        fire(0, idxA, wA, rowsA, semA)

        def pair_body(i, carry):
            g0 = 2 * i
            fire(g0 + 1, idxB, wB, rowsB, semB)
            process(g0, wA, rowsA, semA, oA)

            @pl.when(i < _NCHUNK // 2 - 1)
            def _():
                fire(g0 + 2, idxA, wA, rowsA, semA)

            process(g0 + 1, wB, rowsB, semB, oB)
            return carry

        lax.fori_loop(0, _NCHUNK // 2, pair_body, 0)
        pltpu.make_async_copy(oA, out_hbm.at[0, 0, pl.ds(0, _CHUNK)], semO).wait()
        pltpu.make_async_copy(oB, out_hbm.at[0, 0, pl.ds(0, _CHUNK)], semO).wait()

    return body(uv, t0, t1, t2, t3)


def _prep_table(m, s):
    return jnp.transpose(m.reshape(_C, s * s))


def kernel(input, mipmap_0, mipmap_1, mipmap_2, mipmap_3):
    uv = jnp.stack([input[..., 0].reshape(_P), input[..., 1].reshape(_P)])
    tables = [_prep_table(m, s)
              for m, s in zip((mipmap_0, mipmap_1, mipmap_2, mipmap_3), _SIZES)]
    out = _sc_sample(uv, *tables)
    return out.transpose(0, 3, 1, 2)


# speedup vs baseline: 1.0032x; 1.0032x over previous
import functools

import jax
import jax.numpy as jnp
from jax import lax
from jax.experimental import pallas as pl
from jax.experimental.pallas import tpu as pltpu
from jax.experimental.pallas import tpu_sc as plsc

_SIZES = (1024, 512, 256, 128)
_C = 16
_B = 4
_HW = 512
_P = _B * _HW * _HW
_NW = 32
_PPW = _P // _NW
_CHUNK = 128
_NCHUNK = _PPW // _CHUNK
_NG = 16


def _sc_sample(uv, t0, t1, t2, t3):
    mesh = plsc.VectorSubcoreMesh(core_axis_name="c", subcore_axis_name="s")

    @functools.partial(
        pl.kernel,
        mesh=mesh,
        out_type=jax.ShapeDtypeStruct((_B, _HW, _HW, _C), jnp.float32),
        compiler_params=pltpu.CompilerParams(use_tc_tiling_on_sc=False),
        scratch_types=[
            pltpu.VMEM((2, _CHUNK), jnp.float32),
            pltpu.VMEM((_NG, _CHUNK), jnp.int32),
            pltpu.VMEM((_NG, _CHUNK), jnp.int32),
            pltpu.VMEM((_NG, _CHUNK), jnp.int32),
            pltpu.VMEM((_NG, _CHUNK), jnp.float32),
            pltpu.VMEM((_NG, _CHUNK), jnp.float32),
            pltpu.VMEM((_NG, _CHUNK), jnp.float32),
            pltpu.VMEM((_NG * _CHUNK, _C), jnp.float32),
            pltpu.VMEM((_NG * _CHUNK, _C), jnp.float32),
            pltpu.VMEM((_NG * _CHUNK, _C), jnp.float32),
            pltpu.VMEM((_CHUNK, _C), jnp.float32),
            pltpu.VMEM((_CHUNK, _C), jnp.float32),
            pltpu.VMEM((_CHUNK, _C), jnp.float32),
            pltpu.SemaphoreType.DMA,
            pltpu.SemaphoreType.DMA,
            pltpu.SemaphoreType.DMA,
            pltpu.SemaphoreType.DMA,
        ],
    )
    def body(uv_hbm, t0_hbm, t1_hbm, t2_hbm, t3_hbm, out_hbm,
             uv_v, idxA, idxB, idxC, wA, wB, wC, rowsA, rowsB, rowsC,
             oA, oB, oC, semA, semB, semC, semO):
        tabs = (t0_hbm, t1_hbm, t2_hbm, t3_hbm)
        wid = lax.axis_index("s") * 2 + lax.axis_index("c")
        wbase = wid * _PPW

        def fire(g, idx_v, w_v, rows_v, sem):
            base = wbase + g * _CHUNK
            pltpu.sync_copy(uv_hbm.at[:, pl.ds(base, _CHUNK)], uv_v)

            def grp_body(gi, c2):
                sl = pl.ds(gi * 16, 16)
                uu = uv_v[0, sl]
                vv = uv_v[1, sl]
                for li, s in enumerate(_SIZES):
                    ix = ((2.0 * uu - 1.0 + 1.0) * s - 1.0) * 0.5
                    iy = ((2.0 * vv - 1.0 + 1.0) * s - 1.0) * 0.5
                    x0i = (ix + 1.0).astype(jnp.int32)
                    y0i = (iy + 1.0).astype(jnp.int32)
                    fx = ix - (x0i.astype(jnp.float32) - 1.0)
                    fy = iy - (y0i.astype(jnp.float32) - 1.0)
                    xc0 = jnp.maximum(x0i - 1, 0)
                    xc1 = jnp.minimum(jnp.maximum(x0i, 0), s - 1)
                    yc0 = jnp.maximum(y0i - 1, 0)
                    yc1 = jnp.minimum(jnp.maximum(y0i, 0), s - 1)
                    w0x = jnp.where(x0i >= 1, 1.0 - fx, 0.0)
                    w1x = jnp.where(x0i <= s - 1, fx, 0.0)
                    w0y = jnp.where(y0i >= 1, 1.0 - fy, 0.0)
                    w1y = jnp.where(y0i <= s - 1, fy, 0.0)
                    r0 = yc0 * s
                    r1 = yc1 * s
                    idx_v[li * 4 + 0, sl] = r0 + xc0
                    idx_v[li * 4 + 1, sl] = r0 + xc1
                    idx_v[li * 4 + 2, sl] = r1 + xc0
                    idx_v[li * 4 + 3, sl] = r1 + xc1
                    w_v[li * 4 + 0, sl] = w0x * w0y
                    w_v[li * 4 + 1, sl] = w1x * w0y
                    w_v[li * 4 + 2, sl] = w0x * w1y
                    w_v[li * 4 + 3, sl] = w1x * w1y
                return c2

            lax.fori_loop(0, _CHUNK // 16, grp_body, 0)
            for li in range(4):
                for c in range(4):
                    k = li * 4 + c
                    pltpu.async_copy(
                        tabs[li].at[idx_v.at[k]],
                        rows_v.at[pl.ds(k * _CHUNK, _CHUNK)], sem)

        def process(g, w_v, rows_v, sem, o_v):
            pltpu.make_async_copy(
                t0_hbm.at[pl.ds(0, _NG * _CHUNK)], rows_v, sem).wait()

            @pl.when(g >= 3)
            def _():
                pltpu.make_async_copy(
                    o_v, out_hbm.at[0, 0, pl.ds(0, _CHUNK)], semO).wait()

            def wgrp_body(gi, c3):
                sl = pl.ds(gi * 16, 16)
                wk = [w_v[k, sl] for k in range(_NG)]
                for j in range(16):
                    p = gi * 16 + j
                    acc = wk[0][j] * rows_v[p]
                    for k in range(1, _NG):
                        acc = acc + wk[k][j] * rows_v[k * _CHUNK + p]
                    o_v[p] = acc
                return c3

            lax.fori_loop(0, _CHUNK // 16, wgrp_body, 0)
            base = wbase + g * _CHUNK
            bb = base // (_HW * _HW)
            yy = (base // _HW) % _HW
            xx = base % _HW
            pltpu.async_copy(o_v, out_hbm.at[bb, yy, pl.ds(xx, _CHUNK)], semO)

        bufs = ((idxA, wA, rowsA, semA, oA),
                (idxB, wB, rowsB, semB, oB),
                (idxC, wC, rowsC, semC, oC))
        fire(0, idxA, wA, rowsA, semA)
        fire(1, idxB, wB, rowsB, semB)

        _NITER = (_NCHUNK + 2) // 3

        def tri_body(i, carry):
            for k in range(3):
                g = 3 * i + k
                bidx, bw, brows, bsem, bo = bufs[k]
                nidx, nw, nrows, nsem, _no = bufs[(k + 2) % 3]

                @pl.when(g + 2 < _NCHUNK)
                def _():
                    fire(g + 2, nidx, nw, nrows, nsem)

                @pl.when(g < _NCHUNK)
                def _():
                    process(g, bw, brows, bsem, bo)
            return carry

        lax.fori_loop(0, _NITER, tri_body, 0)
        pltpu.make_async_copy(oA, out_hbm.at[0, 0, pl.ds(0, _CHUNK)], semO).wait()
        pltpu.make_async_copy(oB, out_hbm.at[0, 0, pl.ds(0, _CHUNK)], semO).wait()
        pltpu.make_async_copy(oC, out_hbm.at[0, 0, pl.ds(0, _CHUNK)], semO).wait()

    return body(uv, t0, t1, t2, t3)


def _prep_table(m, s):
    return jnp.transpose(m.reshape(_C, s * s))


def kernel(input, mipmap_0, mipmap_1, mipmap_2, mipmap_3):
    uv = jnp.stack([input[..., 0].reshape(_P), input[..., 1].reshape(_P)])
    tables = [_prep_table(m, s)
              for m, s in zip((mipmap_0, mipmap_1, mipmap_2, mipmap_3), _SIZES)]
    out = _sc_sample(uv, *tables)
    return out.transpose(0, 3, 1, 2)
